# Initial kernel scaffold; baseline (speedup 1.0000x reference)
import jax
import jax.numpy as jnp
from jax.experimental import pallas as pl


def kernel(x, W_in, b_in, codebook, W_out, b_out):
    raise NotImplementedError("write your pallas kernel here")


# speedup vs baseline: 1.4214x; 1.4214x over previous
import functools

import jax
import jax.numpy as jnp
from jax import lax
from jax.experimental import pallas as pl
from jax.experimental.pallas import tpu as pltpu
from jax.experimental.pallas import tpu_sc as plsc

TOK_BLK = 1024


def _tc_body(x_ref, w_in_ref, b_in_ref, cbt_ref, cb_ref, w_out_ref,
             b_out_ref, idx_ref, table_ref):
    h = jnp.dot(x_ref[...], w_in_ref[...],
                preferred_element_type=jnp.float32) + b_in_ref[...]
    s = jnp.dot(h, cbt_ref[...], preferred_element_type=jnp.float32)
    b2 = jnp.sum(cbt_ref[...] * cbt_ref[...], axis=0, keepdims=True)
    scores = b2 - 2.0 * s
    idx_ref[...] = jnp.argmin(scores, axis=1).astype(jnp.int32)

    @pl.when(pl.program_id(0) == 0)
    def _():
        table_ref[...] = jnp.dot(cb_ref[...], w_out_ref[...],
                                 preferred_element_type=jnp.float32
                                 ) + b_out_ref[...]


def _tc_argmin(xf, W_in, b_in, codebook, W_out_p, b_out_p):
    n, in_dim = xf.shape
    k, latent = codebook.shape
    dp = W_out_p.shape[1]
    grid = (n // TOK_BLK,)
    return pl.pallas_call(
        _tc_body,
        grid=grid,
        in_specs=[
            pl.BlockSpec((TOK_BLK, in_dim), lambda i: (i, 0)),
            pl.BlockSpec((in_dim, latent), lambda i: (0, 0)),
            pl.BlockSpec((1, latent), lambda i: (0, 0)),
            pl.BlockSpec((latent, k), lambda i: (0, 0)),
            pl.BlockSpec((k, latent), lambda i: (0, 0)),
            pl.BlockSpec((latent, dp), lambda i: (0, 0)),
            pl.BlockSpec((1, dp), lambda i: (0, 0)),
        ],
        out_specs=[
            pl.BlockSpec((TOK_BLK,), lambda i: (i,)),
            pl.BlockSpec((k, dp), lambda i: (0, 0)),
        ],
        out_shape=[
            jax.ShapeDtypeStruct((n,), jnp.int32),
            jax.ShapeDtypeStruct((k, dp), jnp.float32),
        ],
    )(xf, W_in, b_in.reshape(1, latent), codebook.T, codebook, W_out_p,
      b_out_p.reshape(1, dp))


def _sc_gather(table, idx3, d_out):
    nw, n_ch, ch = idx3.shape
    k, dp = table.shape
    b = nw * n_ch * ch
    mesh = plsc.VectorSubcoreMesh(core_axis_name="c", subcore_axis_name="s")

    @functools.partial(
        pl.kernel,
        mesh=mesh,
        out_type=jax.ShapeDtypeStruct((b, d_out), jnp.float32),
        scratch_types=[
            pltpu.VMEM((n_ch, ch), jnp.int32),
            pltpu.VMEM((ch, d_out), jnp.float32),
            pltpu.SemaphoreType.DMA,
        ],
        compiler_params=pltpu.CompilerParams(use_tc_tiling_on_sc=False),
    )
    def gather(table_hbm, idx_hbm, out_hbm, idx_v, rows_v, sem):
        wid = lax.axis_index("s") * 2 + lax.axis_index("c")
        base = wid * (n_ch * ch)
        pltpu.sync_copy(idx_hbm.at[wid], idx_v)

        def chunk(c, carry):
            pltpu.async_copy(table_hbm.at[idx_v.at[c]], rows_v, sem).wait()
            pltpu.sync_copy(rows_v, out_hbm.at[pl.ds(base + c * ch, ch)])
            return carry

        lax.fori_loop(0, n_ch, chunk, 0)

    return gather(table, idx3)


def kernel(x, W_in, b_in, codebook, W_out, b_out):
    bsz, seq, in_dim = x.shape
    n = bsz * seq
    xf = x.reshape(n, in_dim)
    idx_flat, table = _tc_argmin(xf, W_in, b_in, codebook, W_out, b_out)
    nw, ch = 32, 128
    idx3 = idx_flat.reshape(nw, n // (nw * ch), ch)
    out = _sc_gather(table, idx3, in_dim)
    return out.reshape(bsz, seq, in_dim), idx_flat.reshape(bsz, seq, 1)


# speedup vs baseline: 1.4222x; 1.0005x over previous
import functools

import jax
import jax.numpy as jnp
from jax import lax
from jax.experimental import pallas as pl
from jax.experimental.pallas import tpu as pltpu
from jax.experimental.pallas import tpu_sc as plsc

TOK_BLK = 1024
RT = 8


def _tc_body(x_ref, w_in_ref, b_in_ref, cbt_ref, cb_ref, w_out_ref,
             b_out_ref, idx_ref, table_ref):
    blk = x_ref.shape[0]
    latent, k = cbt_ref.shape
    h = jnp.dot(x_ref[...], w_in_ref[...],
                preferred_element_type=jnp.float32) + b_in_ref[...]
    s = jnp.dot(h, cbt_ref[...], preferred_element_type=jnp.float32)
    b2 = jnp.sum(cbt_ref[...] * cbt_ref[...], axis=0, keepdims=True)
    scores = b2 - 2.0 * s
    idx_ref[0] = jnp.argmin(scores, axis=1).astype(jnp.int32).reshape(1, blk)

    @pl.when(pl.program_id(0) == 0)
    def _():
        table_ref[...] = jnp.dot(cb_ref[...], w_out_ref[...],
                                 preferred_element_type=jnp.float32
                                 ) + b_out_ref[...]


def _tc_argmin(xf, W_in, b_in, codebook, W_out, b_out):
    n, in_dim = xf.shape
    k, latent = codebook.shape
    grid = (n // TOK_BLK,)
    return pl.pallas_call(
        _tc_body,
        grid=grid,
        in_specs=[
            pl.BlockSpec((TOK_BLK, in_dim), lambda i: (i, 0)),
            pl.BlockSpec((in_dim, latent), lambda i: (0, 0)),
            pl.BlockSpec((1, latent), lambda i: (0, 0)),
            pl.BlockSpec((latent, k), lambda i: (0, 0)),
            pl.BlockSpec((k, latent), lambda i: (0, 0)),
            pl.BlockSpec((latent, in_dim), lambda i: (0, 0)),
            pl.BlockSpec((1, in_dim), lambda i: (0, 0)),
        ],
        out_specs=[
            pl.BlockSpec((1, 1, TOK_BLK), lambda i: (i, 0, 0)),
            pl.BlockSpec((k, in_dim), lambda i: (0, 0)),
        ],
        out_shape=[
            jax.ShapeDtypeStruct((n // TOK_BLK, 1, TOK_BLK), jnp.int32),
            jax.ShapeDtypeStruct((k, in_dim), jnp.float32),
        ],
    )(xf, W_in, b_in.reshape(1, latent), codebook.T, codebook, W_out,
      b_out.reshape(1, in_dim))


def _sc_gather(table, idx3, d_out):
    nw, n_ch, ch = idx3.shape
    b = nw * n_ch * ch
    mesh = plsc.VectorSubcoreMesh(core_axis_name="c", subcore_axis_name="s")

    @functools.partial(
        pl.kernel,
        mesh=mesh,
        out_type=jax.ShapeDtypeStruct((b, d_out), jnp.float32),
        scratch_types=[
            pltpu.VMEM((n_ch, ch), jnp.int32),
            pltpu.VMEM((ch, d_out), jnp.float32),
            pltpu.VMEM((ch, d_out), jnp.float32),
            pltpu.SemaphoreType.DMA,
            pltpu.SemaphoreType.DMA,
        ],
        compiler_params=pltpu.CompilerParams(use_tc_tiling_on_sc=False),
    )
    def gather(table_hbm, idx_hbm, out_hbm, idx_v, rows0, rows1, sem0, sem1):
        wid = lax.axis_index("s") * 2 + lax.axis_index("c")
        base = wid * (n_ch * ch)
        pltpu.sync_copy(idx_hbm.at[wid], idx_v)
        rows = (rows0, rows1)
        sems = (sem0, sem1)

        def gcopy(c, bi):
            return pltpu.make_async_copy(table_hbm.at[idx_v.at[c]],
                                         rows[bi], sems[bi])

        gcopy(0, 0).start()

        def body(g2, carry):
            for bi in (0, 1):
                c = g2 * 2 + bi

                @pl.when(c + 1 < n_ch)
                def _():
                    gcopy(c + 1, (bi + 1) % 2).start()

                gcopy(c, bi).wait()
                pltpu.sync_copy(rows[bi],
                                out_hbm.at[pl.ds(base + c * ch, ch)])
            return carry

        lax.fori_loop(0, n_ch // 2, body, 0)

    return gather(table, idx3)


def kernel(x, W_in, b_in, codebook, W_out, b_out):
    bsz, seq, in_dim = x.shape
    n = bsz * seq
    xf = x.reshape(n, in_dim)
    idx3tc, table = _tc_argmin(xf, W_in, b_in, codebook, W_out, b_out)
    idx_flat = idx3tc.reshape(n)
    nw, ch = 32, 128
    out = _sc_gather(table, idx_flat.reshape(nw, n // (nw * ch), ch), in_dim)
    return out.reshape(bsz, seq, in_dim), idx_flat.reshape(bsz, seq, 1)


# speedup vs baseline: 1.8299x; 1.2867x over previous
import functools

import jax
import jax.numpy as jnp
from jax import lax
from jax.experimental import pallas as pl
from jax.experimental.pallas import tpu as pltpu
from jax.experimental.pallas import tpu_sc as plsc

TOK_BLK = 1024
RT = 8


def _tc_body(xt_ref, w_in_t_ref, b_in_ref, cb_ref, w_out_ref, b_out_ref,
             idx_ref, table_ref, st_ref):
    blk = xt_ref.shape[-1]
    k, latent = cb_ref.shape
    ht = jnp.dot(w_in_t_ref[...], xt_ref[0],
                 preferred_element_type=jnp.float32) + b_in_ref[...]
    st = jnp.dot(cb_ref[...], ht, preferred_element_type=jnp.float32)
    b2half = 0.5 * jnp.sum(cb_ref[...] * cb_ref[...], axis=1, keepdims=True)
    st_ref[...] = st - b2half

    sub_iota = lax.broadcasted_iota(jnp.int32, (RT, blk), 0)

    def scan_rt(r, carry):
        best_v, best_i = carry
        v = st_ref[pl.ds(r * RT, RT), :]
        i = sub_iota + r * RT
        take = v > best_v
        return (jnp.where(take, v, best_v), jnp.where(take, i, best_i))

    best_v, best_i = lax.fori_loop(
        1, k // RT, scan_rt, (st_ref[pl.ds(0, RT), :], sub_iota), unroll=8)
    m = jnp.max(best_v, axis=0, keepdims=True)
    idxc = jnp.where(best_v == m, best_i, k)
    idx_ref[0] = jnp.min(idxc, axis=0, keepdims=True).astype(jnp.int32)

    @pl.when(pl.program_id(0) == 0)
    def _():
        table_ref[...] = jnp.dot(cb_ref[...], w_out_ref[...],
                                 preferred_element_type=jnp.float32
                                 ) + b_out_ref[...]


def _tc_argmin(xt, W_in_t, b_in, codebook, W_out, b_out):
    nblk, in_dim, blk = xt.shape
    assert blk == TOK_BLK
    k, latent = codebook.shape
    n = nblk * blk
    return pl.pallas_call(
        _tc_body,
        grid=(nblk,),
        in_specs=[
            pl.BlockSpec((1, in_dim, TOK_BLK), lambda i: (i, 0, 0)),
            pl.BlockSpec((latent, in_dim), lambda i: (0, 0)),
            pl.BlockSpec((latent, 1), lambda i: (0, 0)),
            pl.BlockSpec((k, latent), lambda i: (0, 0)),
            pl.BlockSpec((latent, in_dim), lambda i: (0, 0)),
            pl.BlockSpec((1, in_dim), lambda i: (0, 0)),
        ],
        out_specs=[
            pl.BlockSpec((1, 1, TOK_BLK), lambda i: (i, 0, 0)),
            pl.BlockSpec((k, in_dim), lambda i: (0, 0)),
        ],
        out_shape=[
            jax.ShapeDtypeStruct((nblk, 1, TOK_BLK), jnp.int32),
            jax.ShapeDtypeStruct((k, in_dim), jnp.float32),
        ],
        scratch_shapes=[
            pltpu.VMEM((k, TOK_BLK), jnp.float32),
        ],
    )(xt, W_in_t, b_in.reshape(latent, 1), codebook, W_out,
      b_out.reshape(1, in_dim))


def _sc_gather(table, idx3, d_out):
    nw, n_ch, ch = idx3.shape
    b = nw * n_ch * ch
    mesh = plsc.VectorSubcoreMesh(core_axis_name="c", subcore_axis_name="s")

    @functools.partial(
        pl.kernel,
        mesh=mesh,
        out_type=jax.ShapeDtypeStruct((b, d_out), jnp.float32),
        scratch_types=[
            pltpu.VMEM((n_ch, ch), jnp.int32),
            pltpu.VMEM((ch, d_out), jnp.float32),
            pltpu.VMEM((ch, d_out), jnp.float32),
            pltpu.SemaphoreType.DMA,
            pltpu.SemaphoreType.DMA,
        ],
        compiler_params=pltpu.CompilerParams(use_tc_tiling_on_sc=False),
    )
    def gather(table_hbm, idx_hbm, out_hbm, idx_v, rows0, rows1, sem0, sem1):
        wid = lax.axis_index("s") * 2 + lax.axis_index("c")
        base = wid * (n_ch * ch)
        pltpu.sync_copy(idx_hbm.at[wid], idx_v)
        rows = (rows0, rows1)
        sems = (sem0, sem1)

        def gcopy(c, bi):
            return pltpu.make_async_copy(table_hbm.at[idx_v.at[c]],
                                         rows[bi], sems[bi])

        gcopy(0, 0).start()

        def body(g2, carry):
            for bi in (0, 1):
                c = g2 * 2 + bi

                @pl.when(c + 1 < n_ch)
                def _():
                    gcopy(c + 1, (bi + 1) % 2).start()

                gcopy(c, bi).wait()
                pltpu.sync_copy(rows[bi],
                                out_hbm.at[pl.ds(base + c * ch, ch)])
            return carry

        lax.fori_loop(0, n_ch // 2, body, 0)

    return gather(table, idx3)


def kernel(x, W_in, b_in, codebook, W_out, b_out):
    bsz, seq, in_dim = x.shape
    n = bsz * seq
    xt = jnp.swapaxes(x, 1, 2).reshape(n // TOK_BLK, in_dim, TOK_BLK)
    idx3tc, table = _tc_argmin(xt, W_in.T, b_in, codebook, W_out, b_out)
    idx_flat = idx3tc.reshape(n)
    nw, ch = 32, 128
    out = _sc_gather(table, idx_flat.reshape(nw, n // (nw * ch), ch), in_dim)
    return out.reshape(bsz, seq, in_dim), idx_flat.reshape(bsz, seq, 1)


# speedup vs baseline: 1.8412x; 1.0062x over previous
import functools

import jax
import jax.numpy as jnp
from jax import lax
from jax.experimental import pallas as pl
from jax.experimental.pallas import tpu as pltpu
from jax.experimental.pallas import tpu_sc as plsc

TOK_BLK = 1024
RT = 8


def _tc_body(xt_ref, w_in_t_ref, b_in_ref, cb_ref, w_out_ref, b_out_ref,
             idx_ref, table_ref, st_ref):
    blk = xt_ref.shape[-1]
    k, latent = cb_ref.shape
    ht = jnp.dot(w_in_t_ref[...], xt_ref[0],
                 preferred_element_type=jnp.float32) + b_in_ref[...]
    st = jnp.dot(cb_ref[...], ht, preferred_element_type=jnp.float32)
    b2half = 0.5 * jnp.sum(cb_ref[...] * cb_ref[...], axis=1, keepdims=True)
    st_ref[...] = st - b2half

    sub_iota = lax.broadcasted_iota(jnp.int32, (RT, blk), 0)

    def scan_rt(r, carry):
        best_v, best_i = carry
        v = st_ref[pl.ds(r * RT, RT), :]
        i = sub_iota + r * RT
        take = v > best_v
        return (jnp.where(take, v, best_v), jnp.where(take, i, best_i))

    best_v, best_i = lax.fori_loop(
        1, k // RT, scan_rt, (st_ref[pl.ds(0, RT), :], sub_iota), unroll=8)
    m = jnp.max(best_v, axis=0, keepdims=True)
    idxc = jnp.where(best_v == m, best_i, k)
    idx_ref[0] = jnp.min(idxc, axis=0, keepdims=True).astype(jnp.int32)

    @pl.when(pl.program_id(0) == 0)
    def _():
        table_ref[...] = jnp.dot(cb_ref[...], w_out_ref[...],
                                 preferred_element_type=jnp.float32
                                 ) + b_out_ref[...]


def _tc_argmin(xt, W_in_t, b_in, codebook, W_out, b_out):
    nblk, in_dim, blk = xt.shape
    assert blk == TOK_BLK
    k, latent = codebook.shape
    n = nblk * blk
    return pl.pallas_call(
        _tc_body,
        grid=(nblk,),
        in_specs=[
            pl.BlockSpec((1, in_dim, TOK_BLK), lambda i: (i, 0, 0)),
            pl.BlockSpec((latent, in_dim), lambda i: (0, 0)),
            pl.BlockSpec((latent, 1), lambda i: (0, 0)),
            pl.BlockSpec((k, latent), lambda i: (0, 0)),
            pl.BlockSpec((latent, in_dim), lambda i: (0, 0)),
            pl.BlockSpec((1, in_dim), lambda i: (0, 0)),
        ],
        out_specs=[
            pl.BlockSpec((1, 1, TOK_BLK), lambda i: (i, 0, 0)),
            pl.BlockSpec((k, in_dim), lambda i: (0, 0)),
        ],
        out_shape=[
            jax.ShapeDtypeStruct((nblk, 1, TOK_BLK), jnp.int32),
            jax.ShapeDtypeStruct((k, in_dim), jnp.float32),
        ],
        scratch_shapes=[
            pltpu.VMEM((k, TOK_BLK), jnp.float32),
        ],
    )(xt, W_in_t, b_in.reshape(latent, 1), codebook, W_out,
      b_out.reshape(1, in_dim))


def _sc_gather(table, idx2, d_out):
    nw, per_w = idx2.shape
    rows_per_slab = 512
    n_sl = per_w // rows_per_slab
    b = nw * per_w
    mesh = plsc.VectorSubcoreMesh(core_axis_name="c", subcore_axis_name="s")

    @functools.partial(
        pl.kernel,
        mesh=mesh,
        out_type=jax.ShapeDtypeStruct((b, d_out), jnp.float32),
        scratch_types=[
            pltpu.VMEM((per_w,), jnp.int32),
            pltpu.VMEM((rows_per_slab, d_out), jnp.float32),
            pltpu.VMEM((rows_per_slab, d_out), jnp.float32),
            pltpu.SemaphoreType.DMA,
            pltpu.SemaphoreType.DMA,
        ],
        compiler_params=pltpu.CompilerParams(use_tc_tiling_on_sc=False),
    )
    def gather(table_hbm, idx_hbm, out_hbm, idx_v, rows0, rows1, sem0, sem1):
        wid = lax.axis_index("s") * 2 + lax.axis_index("c")
        base = wid * per_w
        pltpu.sync_copy(idx_hbm.at[wid], idx_v)
        rows = (rows0, rows1)
        sems = (sem0, sem1)

        def gcopy(c, bi):
            return pltpu.make_async_copy(
                table_hbm.at[idx_v.at[pl.ds(c * rows_per_slab,
                                            rows_per_slab)]],
                rows[bi], sems[bi])

        gcopy(0, 0).start()

        def body(g2, carry):
            for bi in (0, 1):
                c = g2 * 2 + bi

                @pl.when(c + 1 < n_sl)
                def _():
                    gcopy(c + 1, (bi + 1) % 2).start()

                gcopy(c, bi).wait()
                pltpu.sync_copy(
                    rows[bi],
                    out_hbm.at[pl.ds(base + c * rows_per_slab,
                                     rows_per_slab)])
            return carry

        lax.fori_loop(0, n_sl // 2, body, 0)

    return gather(table, idx2)


def kernel(x, W_in, b_in, codebook, W_out, b_out):
    bsz, seq, in_dim = x.shape
    n = bsz * seq
    xt = jnp.swapaxes(x, 1, 2).reshape(n // TOK_BLK, in_dim, TOK_BLK)
    idx3tc, table = _tc_argmin(xt, W_in.T, b_in, codebook, W_out, b_out)
    idx_flat = idx3tc.reshape(n)
    out = _sc_gather(table, idx_flat.reshape(32, n // 32), in_dim)
    return out.reshape(bsz, seq, in_dim), idx_flat.reshape(bsz, seq, 1)


# speedup vs baseline: 2.8029x; 1.5223x over previous
import functools

import jax
import jax.numpy as jnp
from jax import lax
from jax.experimental import pallas as pl
from jax.experimental.pallas import tpu as pltpu
from jax.experimental.pallas import tpu_sc as plsc

TOK_BLK = 1024
RT = 8


BPS = 2


def _tc_body(xt_ref, w_in_t_ref, b_in_ref, cb_ref, w_out_ref, b_out_ref,
             idx_ref, table_ref, st_ref, b2_ref):
    blk = BPS * xt_ref.shape[-1]
    k, latent = cb_ref.shape
    @pl.when(pl.program_id(0) == 0)
    def _():
        b2_ref[...] = 0.5 * jnp.sum(cb_ref[...] * cb_ref[...], axis=1,
                                    keepdims=True)
        table_ref[...] = jnp.dot(cb_ref[...], w_out_ref[...],
                                 preferred_element_type=jnp.float32
                                 ) + b_out_ref[...]

    ht = jnp.concatenate(
        [jnp.dot(w_in_t_ref[...], xt_ref[i],
                 preferred_element_type=jnp.float32) for i in range(BPS)],
        axis=1) + b_in_ref[...]
    st = jnp.dot(cb_ref[...], ht, preferred_element_type=jnp.float32)
    st_ref[...] = st - b2_ref[...]

    cc_w = 1024
    for cc in range(blk // cc_w):
        sub_iota = lax.broadcasted_iota(jnp.int32, (RT, cc_w), 0)

        def scan_rt(r, carry):
            best_v, best_i = carry
            v = st_ref[pl.ds(r * RT, RT), pl.ds(cc * cc_w, cc_w)]
            i = sub_iota + r * RT
            take = v > best_v
            return (jnp.where(take, v, best_v), jnp.where(take, i, best_i))

        best_v, best_i = lax.fori_loop(
            1, k // RT, scan_rt,
            (st_ref[pl.ds(0, RT), pl.ds(cc * cc_w, cc_w)], sub_iota),
            unroll=8)
        m = jnp.max(best_v, axis=0, keepdims=True)
        idxc = jnp.where(best_v == m, best_i, k)
        idx_ref[0, :, pl.ds(cc * cc_w, cc_w)] = jnp.min(
            idxc, axis=0, keepdims=True).astype(jnp.int32)


def _tc_argmin(xt, W_in_t, b_in, codebook, W_out, b_out):
    nblk, in_dim, seq = xt.shape
    k, latent = codebook.shape
    blk = BPS * seq
    return pl.pallas_call(
        _tc_body,
        grid=(nblk // BPS,),
        in_specs=[
            pl.BlockSpec((BPS, in_dim, seq), lambda i: (i, 0, 0)),
            pl.BlockSpec((latent, in_dim), lambda i: (0, 0)),
            pl.BlockSpec((latent, 1), lambda i: (0, 0)),
            pl.BlockSpec((k, latent), lambda i: (0, 0)),
            pl.BlockSpec((latent, in_dim), lambda i: (0, 0)),
            pl.BlockSpec((1, in_dim), lambda i: (0, 0)),
        ],
        out_specs=[
            pl.BlockSpec((1, 1, blk), lambda i: (i, 0, 0)),
            pl.BlockSpec((k, in_dim), lambda i: (0, 0)),
        ],
        out_shape=[
            jax.ShapeDtypeStruct((nblk // BPS, 1, blk), jnp.int32),
            jax.ShapeDtypeStruct((k, in_dim), jnp.float32),
        ],
        scratch_shapes=[
            pltpu.VMEM((k, blk), jnp.float32),
            pltpu.VMEM((k, 1), jnp.float32),
        ],
    )(xt, W_in_t, b_in.reshape(latent, 1), codebook, W_out,
      b_out.reshape(1, in_dim))


def _sc_gather(table, idx2, d_out):
    nw, per_w = idx2.shape
    rows_per_slab = 512
    n_sl = per_w // rows_per_slab
    b = nw * per_w
    k = table.shape[0]
    mesh = plsc.VectorSubcoreMesh(core_axis_name="c", subcore_axis_name="s")

    @functools.partial(
        pl.kernel,
        mesh=mesh,
        out_type=jax.ShapeDtypeStruct((b, d_out), jnp.float32),
        scratch_types=[
            pltpu.VMEM((per_w,), jnp.int32),
            pltpu.VMEM((rows_per_slab, d_out), jnp.float32),
            pltpu.VMEM((rows_per_slab, d_out), jnp.float32),
            pltpu.VMEM_SHARED((k, d_out), jnp.float32),
            pltpu.SemaphoreType.DMA,
            pltpu.SemaphoreType.DMA,
        ],
        compiler_params=pltpu.CompilerParams(use_tc_tiling_on_sc=False),
    )
    def gather(table_hbm, idx_hbm, out_hbm, idx_v, rows0, rows1, table_sp,
               sem0, sem1):
        wid = lax.axis_index("s") * 2 + lax.axis_index("c")
        base = wid * per_w

        @pl.when(lax.axis_index("s") == 0)
        def _():
            pltpu.sync_copy(table_hbm, table_sp)

        pltpu.sync_copy(idx_hbm.at[wid], idx_v)
        plsc.subcore_barrier()
        rows = (rows0, rows1)
        sems = (sem0, sem1)

        def gcopy(c, bi):
            return pltpu.make_async_copy(
                table_sp.at[idx_v.at[pl.ds(c * rows_per_slab,
                                           rows_per_slab)]],
                rows[bi], sems[bi])

        gcopy(0, 0).start()

        def body(g2, carry):
            for bi in (0, 1):
                c = g2 * 2 + bi

                @pl.when(c + 1 < n_sl)
                def _():
                    gcopy(c + 1, (bi + 1) % 2).start()

                gcopy(c, bi).wait()
                pltpu.sync_copy(
                    rows[bi],
                    out_hbm.at[pl.ds(base + c * rows_per_slab,
                                     rows_per_slab)])
            return carry

        lax.fori_loop(0, n_sl // 2, body, 0)

    return gather(table, idx2)


def kernel(x, W_in, b_in, codebook, W_out, b_out):
    bsz, seq, in_dim = x.shape
    n = bsz * seq
    xt = jnp.swapaxes(x, 1, 2)
    idx3tc, table = _tc_argmin(xt, W_in.T, b_in, codebook, W_out, b_out)
    idx_flat = idx3tc.reshape(n)
    out = _sc_gather(table, idx_flat.reshape(32, n // 32), in_dim)
    return out.reshape(bsz, seq, in_dim), idx_flat.reshape(bsz, seq, 1)


# speedup vs baseline: 2.9274x; 1.0444x over previous
import functools

import jax
import jax.numpy as jnp
from jax import lax
from jax.experimental import pallas as pl
from jax.experimental.pallas import tpu as pltpu
from jax.experimental.pallas import tpu_sc as plsc

TOK_BLK = 1024
RT = 8


BPS = 4


def _tc_body(xt_ref, w_in_t_ref, b_in_ref, cb_ref, w_out_ref, b_out_ref,
             idx_ref, table_ref, st_ref, b2_ref):
    blk = BPS * xt_ref.shape[-1]
    k, latent = cb_ref.shape
    @pl.when(pl.program_id(0) == 0)
    def _():
        b2_ref[...] = 0.5 * jnp.sum(cb_ref[...] * cb_ref[...], axis=1,
                                    keepdims=True)
        table_ref[...] = jnp.dot(cb_ref[...], w_out_ref[...],
                                 preferred_element_type=jnp.float32
                                 ) + b_out_ref[...]

    ht = jnp.concatenate(
        [jnp.dot(w_in_t_ref[...], xt_ref[i],
                 preferred_element_type=jnp.float32) for i in range(BPS)],
        axis=1) + b_in_ref[...]
    st = jnp.dot(cb_ref[...], ht, preferred_element_type=jnp.float32)
    st_ref[...] = st - b2_ref[...]

    cc_w = 1024
    for cc in range(blk // cc_w):
        sub_iota = lax.broadcasted_iota(jnp.int32, (RT, cc_w), 0)

        def scan_rt(r, carry):
            best_v, best_i = carry
            v = st_ref[pl.ds(r * RT, RT), pl.ds(cc * cc_w, cc_w)]
            i = sub_iota + r * RT
            take = v > best_v
            return (jnp.where(take, v, best_v), jnp.where(take, i, best_i))

        best_v, best_i = lax.fori_loop(
            1, k // RT, scan_rt,
            (st_ref[pl.ds(0, RT), pl.ds(cc * cc_w, cc_w)], sub_iota),
            unroll=8)
        m = jnp.max(best_v, axis=0, keepdims=True)
        idxc = jnp.where(best_v == m, best_i, k)
        idx_ref[0, :, pl.ds(cc * cc_w, cc_w)] = jnp.min(
            idxc, axis=0, keepdims=True).astype(jnp.int32)


def _tc_argmin(xt, W_in_t, b_in, codebook, W_out, b_out):
    nblk, in_dim, seq = xt.shape
    k, latent = codebook.shape
    blk = BPS * seq
    return pl.pallas_call(
        _tc_body,
        grid=(nblk // BPS,),
        in_specs=[
            pl.BlockSpec((BPS, in_dim, seq), lambda i: (i, 0, 0)),
            pl.BlockSpec((latent, in_dim), lambda i: (0, 0)),
            pl.BlockSpec((latent, 1), lambda i: (0, 0)),
            pl.BlockSpec((k, latent), lambda i: (0, 0)),
            pl.BlockSpec((latent, in_dim), lambda i: (0, 0)),
            pl.BlockSpec((1, in_dim), lambda i: (0, 0)),
        ],
        out_specs=[
            pl.BlockSpec((1, 1, blk), lambda i: (i, 0, 0)),
            pl.BlockSpec((k, in_dim), lambda i: (0, 0)),
        ],
        out_shape=[
            jax.ShapeDtypeStruct((nblk // BPS, 1, blk), jnp.int32),
            jax.ShapeDtypeStruct((k, in_dim), jnp.float32),
        ],
        scratch_shapes=[
            pltpu.VMEM((k, blk), jnp.float32),
            pltpu.VMEM((k, 1), jnp.float32),
        ],
    )(xt, W_in_t, b_in.reshape(latent, 1), codebook, W_out,
      b_out.reshape(1, in_dim))


def _sc_gather(table, idx2, d_out):
    nw, per_w = idx2.shape
    rows_per_slab = 512
    n_sl = per_w // rows_per_slab
    b = nw * per_w
    k = table.shape[0]
    mesh = plsc.VectorSubcoreMesh(core_axis_name="c", subcore_axis_name="s")

    @functools.partial(
        pl.kernel,
        mesh=mesh,
        out_type=jax.ShapeDtypeStruct((b, d_out), jnp.float32),
        scratch_types=[
            pltpu.VMEM((per_w,), jnp.int32),
            pltpu.VMEM((rows_per_slab, d_out), jnp.float32),
            pltpu.VMEM((rows_per_slab, d_out), jnp.float32),
            pltpu.VMEM_SHARED((k, d_out), jnp.float32),
            pltpu.SemaphoreType.DMA,
            pltpu.SemaphoreType.DMA,
        ],
        compiler_params=pltpu.CompilerParams(use_tc_tiling_on_sc=False),
    )
    def gather(table_hbm, idx_hbm, out_hbm, idx_v, rows0, rows1, table_sp,
               sem0, sem1):
        wid = lax.axis_index("s") * 2 + lax.axis_index("c")
        base = wid * per_w

        @pl.when(lax.axis_index("s") == 0)
        def _():
            pltpu.sync_copy(table_hbm, table_sp)

        pltpu.sync_copy(idx_hbm.at[wid], idx_v)
        plsc.subcore_barrier()
        rows = (rows0, rows1)
        sems = (sem0, sem1)

        def gcopy(c, bi):
            return pltpu.make_async_copy(
                table_sp.at[idx_v.at[pl.ds(c * rows_per_slab,
                                           rows_per_slab)]],
                rows[bi], sems[bi])

        gcopy(0, 0).start()

        def body(g2, carry):
            for bi in (0, 1):
                c = g2 * 2 + bi

                @pl.when(c + 1 < n_sl)
                def _():
                    gcopy(c + 1, (bi + 1) % 2).start()

                gcopy(c, bi).wait()
                pltpu.sync_copy(
                    rows[bi],
                    out_hbm.at[pl.ds(base + c * rows_per_slab,
                                     rows_per_slab)])
            return carry

        lax.fori_loop(0, n_sl // 2, body, 0)

    return gather(table, idx2)


def kernel(x, W_in, b_in, codebook, W_out, b_out):
    bsz, seq, in_dim = x.shape
    n = bsz * seq
    xt = jnp.swapaxes(x, 1, 2)
    idx3tc, table = _tc_argmin(xt, W_in.T, b_in, codebook, W_out, b_out)
    idx_flat = idx3tc.reshape(n)
    out = _sc_gather(table, idx_flat.reshape(32, n // 32), in_dim)
    return out.reshape(bsz, seq, in_dim), idx_flat.reshape(bsz, seq, 1)
